# baseline (speedup 1.0000x reference)
import functools

import jax
import jax.numpy as jnp
from jax import lax
from jax.experimental import pallas as pl
from jax.experimental.pallas import tpu as pltpu
from jax.experimental.pallas import tpu_sc as plsc

B = 4096
D = 128
NVF = 9
HEXB = 16
F = NVF * HEXB
VOCAB = 1000

_NC = 2
_NS = 16
_NW = _NC * _NS
_BPW = B // _NW
_CHUNK = 64
_NCHUNK = _BPW // _CHUNK


def _sc_partial(idx_w, t_sp, t_ab, t_it, t_sd, t_ac):
  mesh = plsc.VectorSubcoreMesh(core_axis_name="c", subcore_axis_name="s")

  @functools.partial(
      pl.kernel,
      mesh=mesh,
      out_type=jax.ShapeDtypeStruct((B, D), jnp.float32),
      scratch_types=[
          pltpu.VMEM((8, _BPW), jnp.int32),
          pltpu.VMEM((8, _CHUNK, D), jnp.float32),
          pltpu.VMEM((_BPW, D), jnp.float32),
          pltpu.VMEM_SHARED((VOCAB, D), jnp.float32),
          pltpu.VMEM_SHARED((VOCAB, D), jnp.float32),
          pltpu.VMEM_SHARED((VOCAB, D), jnp.float32),
          pltpu.VMEM_SHARED((VOCAB, D), jnp.float32),
          pltpu.VMEM_SHARED((2, D), jnp.float32),
          pltpu.SemaphoreType.DMA,
      ],
  )
  def k(idx_hbm, sp, ab, it, sd, ac, out_hbm, idx_v, rbuf, obuf,
        m_sp, m_ab, m_it, m_ac, m_sd, gsem):
    cid = lax.axis_index("c")
    sid = lax.axis_index("s")
    wid = sid * _NC + cid
    base = wid * _BPW
    icps = [
        pltpu.async_copy(idx_hbm.at[j, pl.ds(base, _BPW)], idx_v.at[j], gsem)
        for j in range(8)
    ]
    for j, (hsrc, mdst) in enumerate(
        [(sp, m_sp), (ab, m_ab), (it, m_it), (sd, m_sd), (ac, m_ac)]):
      @pl.when(sid == j)
      def _():
        pltpu.sync_copy(hsrc, mdst)
    for cp in icps:
      cp.wait()
    plsc.subcore_barrier()
    tbls = [m_sp, m_ab, m_it, m_sd, m_ac, m_ac, m_ac, m_ac]
    for ch in range(_NCHUNK):
      off = ch * _CHUNK
      cps = [
          pltpu.async_copy(tbls[j].at[idx_v.at[j, pl.ds(off, _CHUNK)]],
                           rbuf.at[j], gsem)
          for j in range(8)
      ]
      for cp in cps:
        cp.wait()

      def row(e, carry):
        for cb in range(D // 16):
          co = cb * 16
          m = (rbuf[4, e, pl.ds(co, 16)] + rbuf[5, e, pl.ds(co, 16)]
               + rbuf[6, e, pl.ds(co, 16)] + rbuf[7, e, pl.ds(co, 16)])
          v = (jnp.maximum(rbuf[0, e, pl.ds(co, 16)], 0.0)
               + jnp.maximum(rbuf[1, e, pl.ds(co, 16)], 0.0)
               + jnp.maximum(rbuf[2, e, pl.ds(co, 16)], 0.0)
               + jnp.maximum(rbuf[3, e, pl.ds(co, 16)], 0.0)
               + jnp.maximum(m * 0.25, 0.0))
          obuf[off + e, pl.ds(co, 16)] = v
        return carry

      lax.fori_loop(0, _CHUNK, row, 0)
    pltpu.sync_copy(obuf, out_hbm.at[pl.ds(base, _BPW)])

  return k(idx_w, t_sp, t_ab, t_it, t_sd, t_ac)


_BLK = 4096


def _tc_hex(volatiles, w_hex):
  def body(v_ref, wh_ref, o_ref):
    bitpos = lax.broadcasted_iota(jnp.int32, (_BLK, HEXB), 1)
    feats = jnp.concatenate(
        [jnp.bitwise_and(
            lax.shift_right_logical(v_ref[:, f:f + 1], bitpos), 1)
         for f in range(NVF)], axis=1).astype(jnp.float32)
    o_ref[...] = jnp.dot(feats, wh_ref[...],
                         preferred_element_type=jnp.float32)

  return pl.pallas_call(
      body,
      grid=(B // _BLK,),
      in_specs=[
          pl.BlockSpec((_BLK, NVF), lambda i: (i, 0)),
          pl.BlockSpec((F, D), lambda i: (0, 0)),
      ],
      out_specs=pl.BlockSpec((_BLK, D), lambda i: (i, 0)),
      out_shape=jax.ShapeDtypeStruct((B, D), jnp.float32),
  )(volatiles, w_hex)


def _tc_final(partial, hexe, species2d, w_out, b2d):
  def body(part_ref, hx_ref, sp_ref, wo_ref, b_ref, o_ref):
    ssum = part_ref[...] + hx_ref[...]
    out = jnp.maximum(
        jnp.dot(ssum, wo_ref[...], preferred_element_type=jnp.float32)
        + b_ref[...], 0.0)
    o_ref[...] = jnp.where(sp_ref[...] != 0, out, 0.0)

  return pl.pallas_call(
      body,
      grid=(B // _BLK,),
      in_specs=[
          pl.BlockSpec((_BLK, D), lambda i: (i, 0)),
          pl.BlockSpec((_BLK, D), lambda i: (i, 0)),
          pl.BlockSpec((_BLK, 1), lambda i: (i, 0)),
          pl.BlockSpec((D, D), lambda i: (0, 0)),
          pl.BlockSpec((1, D), lambda i: (0, 0)),
      ],
      out_specs=pl.BlockSpec((_BLK, D), lambda i: (i, 0)),
      out_shape=jax.ShapeDtypeStruct((B, D), jnp.float32),
  )(partial, hexe, species2d, w_out, b2d)


def kernel(species_idx, ability_idx, item_idx, side_idx, move_ids, volatiles,
           species_table, abilities_table, items_table, actions_table,
           side_table, W_hex, W_out, b_out):
  sp = species_idx.astype(jnp.int32)
  idx_all = jnp.stack([
      sp,
      ability_idx.astype(jnp.int32),
      item_idx.astype(jnp.int32),
      side_idx.astype(jnp.int32),
      move_ids[:, 0].astype(jnp.int32),
      move_ids[:, 1].astype(jnp.int32),
      move_ids[:, 2].astype(jnp.int32),
      move_ids[:, 3].astype(jnp.int32),
  ])
  partial = _sc_partial(idx_all, species_table, abilities_table, items_table,
                        side_table, actions_table)
  hexe = _tc_hex(volatiles.astype(jnp.int32), W_hex)
  return _tc_final(partial, hexe, sp[:, None], W_out, b_out[None, :])


# speedup vs baseline: 1.0285x; 1.0285x over previous
import functools

import jax
import jax.numpy as jnp
from jax import lax
from jax.experimental import pallas as pl
from jax.experimental.pallas import tpu as pltpu
from jax.experimental.pallas import tpu_sc as plsc

B = 4096
D = 128
NVF = 9
HEXB = 16
F = NVF * HEXB
VOCAB = 1000

_NC = 2
_NS = 16
_NW = _NC * _NS
_BPW = B // _NW
_CHUNK = 64
_NCHUNK = _BPW // _CHUNK


def _sc_partial(idx_w, t_sp, t_ab, t_it, t_sd, t_ac):
  mesh = plsc.VectorSubcoreMesh(core_axis_name="c", subcore_axis_name="s")

  @functools.partial(
      pl.kernel,
      mesh=mesh,
      out_type=jax.ShapeDtypeStruct((B, D), jnp.float32),
      scratch_types=[
          pltpu.VMEM((8, _BPW), jnp.int32),
          pltpu.VMEM((8, _CHUNK, D), jnp.float32),
          pltpu.VMEM((_BPW, D), jnp.float32),
          pltpu.VMEM_SHARED((VOCAB, D), jnp.float32),
          pltpu.VMEM_SHARED((VOCAB, D), jnp.float32),
          pltpu.VMEM_SHARED((VOCAB, D), jnp.float32),
          pltpu.VMEM_SHARED((VOCAB, D), jnp.float32),
          pltpu.VMEM_SHARED((2, D), jnp.float32),
          pltpu.SemaphoreType.DMA,
      ],
  )
  def k(idx_hbm, sp, ab, it, sd, ac, out_hbm, idx_v, rbuf, obuf,
        m_sp, m_ab, m_it, m_ac, m_sd, gsem):
    cid = lax.axis_index("c")
    sid = lax.axis_index("s")
    wid = sid * _NC + cid
    base = wid * _BPW
    icps = [
        pltpu.async_copy(idx_hbm.at[j, pl.ds(base, _BPW)], idx_v.at[j], gsem)
        for j in range(8)
    ]
    for j, (hsrc, mdst) in enumerate(
        [(sp, m_sp), (ab, m_ab), (it, m_it), (sd, m_sd), (ac, m_ac)]):
      @pl.when(sid == j)
      def _():
        pltpu.sync_copy(hsrc, mdst)
    for cp in icps:
      cp.wait()
    plsc.subcore_barrier()
    tbls = [m_sp, m_ab, m_it, m_sd, m_ac, m_ac, m_ac, m_ac]
    for ch in range(_NCHUNK):
      off = ch * _CHUNK
      cps = [
          pltpu.async_copy(tbls[j].at[idx_v.at[j, pl.ds(off, _CHUNK)]],
                           rbuf.at[j], gsem)
          for j in range(8)
      ]
      for cp in cps:
        cp.wait()

      def row(e, carry):
        for cb in range(D // 16):
          co = cb * 16
          m = (rbuf[4, e, pl.ds(co, 16)] + rbuf[5, e, pl.ds(co, 16)]
               + rbuf[6, e, pl.ds(co, 16)] + rbuf[7, e, pl.ds(co, 16)])
          v = (jnp.maximum(rbuf[0, e, pl.ds(co, 16)], 0.0)
               + jnp.maximum(rbuf[1, e, pl.ds(co, 16)], 0.0)
               + jnp.maximum(rbuf[2, e, pl.ds(co, 16)], 0.0)
               + jnp.maximum(rbuf[3, e, pl.ds(co, 16)], 0.0)
               + jnp.maximum(m * 0.25, 0.0))
          obuf[off + e, pl.ds(co, 16)] = v
        return carry

      lax.fori_loop(0, _CHUNK, row, 0)
    pltpu.sync_copy(obuf, out_hbm.at[pl.ds(base, _BPW)])

  return k(idx_w, t_sp, t_ab, t_it, t_sd, t_ac)


_BLK = 2048


def _tc_hex(volatiles, w_hex):
  def body(v_ref, wh_ref, o_ref):
    bitpos = lax.broadcasted_iota(jnp.int32, (_BLK, HEXB), 1)
    feats = jnp.concatenate(
        [jnp.bitwise_and(
            lax.shift_right_logical(v_ref[:, f:f + 1], bitpos), 1)
         for f in range(NVF)], axis=1).astype(jnp.float32)
    o_ref[...] = jnp.dot(feats, wh_ref[...],
                         preferred_element_type=jnp.float32)

  return pl.pallas_call(
      body,
      grid=(B // _BLK,),
      in_specs=[
          pl.BlockSpec((_BLK, NVF), lambda i: (i, 0)),
          pl.BlockSpec((F, D), lambda i: (0, 0)),
      ],
      out_specs=pl.BlockSpec((_BLK, D), lambda i: (i, 0)),
      out_shape=jax.ShapeDtypeStruct((B, D), jnp.float32),
  )(volatiles, w_hex)


def _tc_final(partial, hexe, species2d, w_out, b2d):
  def body(part_ref, hx_ref, sp_ref, wo_ref, b_ref, o_ref):
    ssum = part_ref[...] + hx_ref[...]
    out = jnp.maximum(
        jnp.dot(ssum, wo_ref[...], preferred_element_type=jnp.float32)
        + b_ref[...], 0.0)
    o_ref[...] = jnp.where(sp_ref[...] != 0, out, 0.0)

  return pl.pallas_call(
      body,
      grid=(B // _BLK,),
      in_specs=[
          pl.BlockSpec((_BLK, D), lambda i: (i, 0)),
          pl.BlockSpec((_BLK, D), lambda i: (i, 0)),
          pl.BlockSpec((_BLK, 1), lambda i: (i, 0)),
          pl.BlockSpec((D, D), lambda i: (0, 0)),
          pl.BlockSpec((1, D), lambda i: (0, 0)),
      ],
      out_specs=pl.BlockSpec((_BLK, D), lambda i: (i, 0)),
      out_shape=jax.ShapeDtypeStruct((B, D), jnp.float32),
  )(partial, hexe, species2d, w_out, b2d)


def kernel(species_idx, ability_idx, item_idx, side_idx, move_ids, volatiles,
           species_table, abilities_table, items_table, actions_table,
           side_table, W_hex, W_out, b_out):
  sp = species_idx.astype(jnp.int32)
  idx_all = jnp.stack([
      sp,
      ability_idx.astype(jnp.int32),
      item_idx.astype(jnp.int32),
      side_idx.astype(jnp.int32),
      move_ids[:, 0].astype(jnp.int32),
      move_ids[:, 1].astype(jnp.int32),
      move_ids[:, 2].astype(jnp.int32),
      move_ids[:, 3].astype(jnp.int32),
  ])
  partial = _sc_partial(idx_all, species_table, abilities_table, items_table,
                        side_table, actions_table)
  hexe = _tc_hex(volatiles.astype(jnp.int32), W_hex)
  return _tc_final(partial, hexe, sp[:, None], W_out, b_out[None, :])
